# baseline (speedup 1.0000x reference)
import jax
import jax.numpy as jnp
from jax import lax
from jax.experimental import pallas as pl
from jax.experimental.pallas import tpu as pltpu
from jax.experimental.pallas import tpu_sc as plsc

V = 100000
F = 200000
NC = 2
NS = 16
NW = NC * NS
CHUNK = 128
FP = 204800
CPW = FP // (NW * CHUNK)
VP = 100352
TPS = VP // NS


def _sc_body(faces_ref, verts_ref, zeros_ref, part_ref,
             accx, accy, accz,
             idx_v, g_v, fn_v, gsem, ssem):
    c = lax.axis_index("c")
    s = lax.axis_index("s")
    wid = c * NS + s
    base = s * TPS

    sl = pl.ds(base, TPS)
    pltpu.sync_copy(zeros_ref, accx.at[sl])
    pltpu.sync_copy(zeros_ref, accy.at[sl])
    pltpu.sync_copy(zeros_ref, accz.at[sl])

    for d in range(3):
        pltpu.sync_copy(faces_ref.at[wid, d], idx_v.at[d])

    plsc.subcore_barrier()

    tabs = tuple(verts_ref.at[d, 0] for d in range(3))
    accs = (accx, accy, accz)

    def gather_cps(j, p):
        cps = []
        for k in range(3):
            idx = idx_v.at[k, j]
            for d in range(3):
                cps.append(pltpu.make_async_copy(
                    tabs[d].at[idx], g_v.at[p, k * 3 + d], gsem.at[p]))
        return cps

    def scatter_cps(j, p):
        cps = []
        for k in range(3):
            idx = idx_v.at[k, j]
            for d in range(3):
                cps.append(pltpu.make_async_copy(
                    fn_v.at[p, d], accs[d].at[idx], ssem.at[p]))
        return cps

    def compute(p):
        for i in range(CHUNK // 16):
            t = pl.ds(16 * i, 16)
            ax = g_v[p, 0, t]
            ay = g_v[p, 1, t]
            az = g_v[p, 2, t]
            bx = g_v[p, 3, t]
            by = g_v[p, 4, t]
            bz = g_v[p, 5, t]
            cx = g_v[p, 6, t]
            cy = g_v[p, 7, t]
            cz = g_v[p, 8, t]
            e0x = bx - ax
            e0y = by - ay
            e0z = bz - az
            e1x = cx - ax
            e1y = cy - ay
            e1z = cz - az
            fn_v[p, 0, t] = e0y * e1z - e0z * e1y
            fn_v[p, 1, t] = e0z * e1x - e0x * e1z
            fn_v[p, 2, t] = e0x * e1y - e0y * e1x

    for cp in gather_cps(0, 0):
        cp.start()

    @pl.loop(0, CPW // 2)
    def _pair(i):
        for half in range(2):
            j = 2 * i + half
            p = half

            @pl.when(j + 1 < CPW)
            def _fire_next_gathers(j=j, p=p):
                for cp in gather_cps(j + 1, 1 - p):
                    cp.start()

            for cp in gather_cps(j, p):
                cp.wait()

            @pl.when(j >= 2)
            def _drain_prev_scatters(j=j, p=p):
                for cp in scatter_cps(j, p):
                    cp.wait()

            compute(p)
            for cp in scatter_cps(j, p):
                cp.start(add=True)

    for cp in scatter_cps(CPW - 2, 0):
        cp.wait()
    for cp in scatter_cps(CPW - 1, 1):
        cp.wait()

    plsc.subcore_barrier()

    pltpu.sync_copy(accx.at[sl], part_ref.at[c, 0, 0, sl])
    pltpu.sync_copy(accy.at[sl], part_ref.at[c, 1, 0, sl])
    pltpu.sync_copy(accz.at[sl], part_ref.at[c, 2, 0, sl])


def _tc_finish_body(p_ref, o_ref):
    p = p_ref[...]
    vn = p[0] + p[1]
    sq = jnp.sum(vn * vn, axis=0, keepdims=True)
    norm = jnp.sqrt(sq)
    normalized = vn / jnp.maximum(norm, 1e-12)
    mask = sq > 1e-20
    default = jnp.where(
        lax.broadcasted_iota(jnp.int32, (3, 1), 0) == 2, 1.0, 0.0
    ).astype(jnp.float32)
    o_ref[...] = jnp.where(mask, normalized, default)


@jax.jit
def kernel(verts, faces):
    vertsT = jnp.zeros((3, 1, VP), jnp.float32).at[:, 0, :V].set(
        jnp.transpose(verts)
    )
    facesT = jnp.transpose(faces).astype(jnp.int32)
    facesP = (
        jnp.concatenate([facesT, jnp.zeros((3, FP - F), jnp.int32)], axis=1)
        .reshape(3, NW, CPW, CHUNK)
        .transpose(1, 0, 2, 3)
    )
    zeros = jnp.zeros((TPS,), jnp.float32)

    mesh = plsc.VectorSubcoreMesh(
        core_axis_name="c", subcore_axis_name="s",
        num_cores=NC, num_subcores=NS,
    )
    partial = pl.kernel(
        _sc_body,
        out_type=jax.ShapeDtypeStruct((NC, 3, 1, VP), jnp.float32),
        mesh=mesh,
        scratch_types=[
            pltpu.VMEM_SHARED((VP,), jnp.float32),
            pltpu.VMEM_SHARED((VP,), jnp.float32),
            pltpu.VMEM_SHARED((VP,), jnp.float32),
            pltpu.VMEM((3, CPW, CHUNK), jnp.int32),
            pltpu.VMEM((2, 9, CHUNK), jnp.float32),
            pltpu.VMEM((2, 3, CHUNK), jnp.float32),
            pltpu.SemaphoreType.DMA((2,)),
            pltpu.SemaphoreType.DMA((2,)),
        ],
    )(facesP, vertsT, zeros)
    partial = partial.reshape(NC, 3, VP)

    B = 2048
    out = pl.pallas_call(
        _tc_finish_body,
        grid=(VP // B,),
        in_specs=[pl.BlockSpec((NC, 3, B), lambda i: (0, 0, i))],
        out_specs=pl.BlockSpec((3, B), lambda i: (0, i)),
        out_shape=jax.ShapeDtypeStruct((3, VP), jnp.float32),
    )(partial)

    return jnp.transpose(out[:, :V])


# speedup vs baseline: 1.3098x; 1.3098x over previous
import jax
import jax.numpy as jnp
from jax import lax
from jax.experimental import pallas as pl
from jax.experimental.pallas import tpu as pltpu
from jax.experimental.pallas import tpu_sc as plsc

V = 100000
F = 200000
NC = 2
NS = 16
NW = NC * NS
CHUNK = 128
FP = 204800
CPW = FP // (NW * CHUNK)
VP = 100352
TPS = VP // NS
NBUF = 5


def _sc_body(faces_ref, verts_ref, zeros_ref, part_ref,
             tabx, taby, tabz, accx, accy, accz,
             idx_v, g_v, fn_v, gsem, ssem):
    c = lax.axis_index("c")
    s = lax.axis_index("s")
    wid = c * NS + s
    base = s * TPS

    sl = pl.ds(base, TPS)
    pltpu.sync_copy(verts_ref.at[0, 0, sl], tabx.at[sl])
    pltpu.sync_copy(verts_ref.at[1, 0, sl], taby.at[sl])
    pltpu.sync_copy(verts_ref.at[2, 0, sl], tabz.at[sl])
    pltpu.sync_copy(zeros_ref, accx.at[sl])
    pltpu.sync_copy(zeros_ref, accy.at[sl])
    pltpu.sync_copy(zeros_ref, accz.at[sl])

    for d in range(3):
        pltpu.sync_copy(faces_ref.at[wid, d], idx_v.at[d])

    plsc.subcore_barrier()

    tabs = (tabx, taby, tabz)
    accs = (accx, accy, accz)

    def gather_cps(j, p):
        cps = []
        for k in range(3):
            idx = idx_v.at[k, j]
            for d in range(3):
                cps.append(pltpu.make_async_copy(
                    tabs[d].at[idx], g_v.at[p, k * 3 + d], gsem.at[p]))
        return cps

    def scatter_cps(j, p):
        cps = []
        for k in range(3):
            idx = idx_v.at[k, j]
            for d in range(3):
                cps.append(pltpu.make_async_copy(
                    fn_v.at[p, d], accs[d].at[idx], ssem.at[p]))
        return cps

    def compute(p):
        for i in range(CHUNK // 16):
            t = pl.ds(16 * i, 16)
            ax = g_v[p, 0, t]
            ay = g_v[p, 1, t]
            az = g_v[p, 2, t]
            bx = g_v[p, 3, t]
            by = g_v[p, 4, t]
            bz = g_v[p, 5, t]
            cx = g_v[p, 6, t]
            cy = g_v[p, 7, t]
            cz = g_v[p, 8, t]
            e0x = bx - ax
            e0y = by - ay
            e0z = bz - az
            e1x = cx - ax
            e1y = cy - ay
            e1z = cz - az
            fn_v[p, 0, t] = e0y * e1z - e0z * e1y
            fn_v[p, 1, t] = e0z * e1x - e0x * e1z
            fn_v[p, 2, t] = e0x * e1y - e0y * e1x

    for jj in range(NBUF - 1):
        for cp in gather_cps(jj, jj):
            cp.start()

    @pl.loop(0, CPW // NBUF)
    def _round(i):
        for half in range(NBUF):
            j = NBUF * i + half
            p = half

            @pl.when(j + NBUF - 1 < CPW)
            def _fire_next_gathers(j=j, p=p):
                for cp in gather_cps(j + NBUF - 1, (p + NBUF - 1) % NBUF):
                    cp.start()

            for cp in gather_cps(j, p):
                cp.wait()

            @pl.when(j >= NBUF)
            def _drain_prev_scatters(j=j, p=p):
                for cp in scatter_cps(j, p):
                    cp.wait()

            compute(p)
            for cp in scatter_cps(j, p):
                cp.start(add=True)

    for jj in range(NBUF):
        for cp in scatter_cps(CPW - NBUF + jj, jj):
            cp.wait()

    plsc.subcore_barrier()

    pltpu.sync_copy(accx.at[sl], part_ref.at[c, 0, 0, sl])
    pltpu.sync_copy(accy.at[sl], part_ref.at[c, 1, 0, sl])
    pltpu.sync_copy(accz.at[sl], part_ref.at[c, 2, 0, sl])


def _tc_finish_body(p_ref, o_ref):
    p = p_ref[...]
    vn = p[0] + p[1]
    sq = jnp.sum(vn * vn, axis=0, keepdims=True)
    norm = jnp.sqrt(sq)
    normalized = vn / jnp.maximum(norm, 1e-12)
    mask = sq > 1e-20
    default = jnp.where(
        lax.broadcasted_iota(jnp.int32, (3, 1), 0) == 2, 1.0, 0.0
    ).astype(jnp.float32)
    o_ref[...] = jnp.where(mask, normalized, default)


@jax.jit
def kernel(verts, faces):
    vertsT = jnp.zeros((3, 1, VP), jnp.float32).at[:, 0, :V].set(
        jnp.transpose(verts)
    )
    facesT = jnp.transpose(faces).astype(jnp.int32)
    facesP = (
        jnp.concatenate([facesT, jnp.zeros((3, FP - F), jnp.int32)], axis=1)
        .reshape(3, NW, CPW, CHUNK)
        .transpose(1, 0, 2, 3)
    )
    zeros = jnp.zeros((TPS,), jnp.float32)

    mesh = plsc.VectorSubcoreMesh(
        core_axis_name="c", subcore_axis_name="s",
        num_cores=NC, num_subcores=NS,
    )
    partial = pl.kernel(
        _sc_body,
        out_type=jax.ShapeDtypeStruct((NC, 3, 1, VP), jnp.float32),
        mesh=mesh,
        scratch_types=[
            pltpu.VMEM_SHARED((VP,), jnp.float32),
            pltpu.VMEM_SHARED((VP,), jnp.float32),
            pltpu.VMEM_SHARED((VP,), jnp.float32),
            pltpu.VMEM_SHARED((VP,), jnp.float32),
            pltpu.VMEM_SHARED((VP,), jnp.float32),
            pltpu.VMEM_SHARED((VP,), jnp.float32),
            pltpu.VMEM((3, CPW, CHUNK), jnp.int32),
            pltpu.VMEM((NBUF, 9, CHUNK), jnp.float32),
            pltpu.VMEM((NBUF, 3, CHUNK), jnp.float32),
            pltpu.SemaphoreType.DMA((NBUF,)),
            pltpu.SemaphoreType.DMA((NBUF,)),
        ],
    )(facesP, vertsT, zeros)
    partial = partial.reshape(NC, 3, VP)

    B = 2048
    out = pl.pallas_call(
        _tc_finish_body,
        grid=(VP // B,),
        in_specs=[pl.BlockSpec((NC, 3, B), lambda i: (0, 0, i))],
        out_specs=pl.BlockSpec((3, B), lambda i: (0, i)),
        out_shape=jax.ShapeDtypeStruct((3, VP), jnp.float32),
    )(partial)

    return jnp.transpose(out[:, :V])


# speedup vs baseline: 1.4656x; 1.1190x over previous
import jax
import jax.numpy as jnp
from jax import lax
from jax.experimental import pallas as pl
from jax.experimental.pallas import tpu as pltpu
from jax.experimental.pallas import tpu_sc as plsc

V = 100000
F = 200000
NC = 2
NS = 16
NW = NC * NS
CHUNK = 128
FP = 204800
CPW = FP // (NW * CHUNK)
VP = 100352
TPS = VP // NS
NBUF = 2


def _sc_body(faces_ref, verts_ref, zeros_ref, part_ref,
             tabx, taby, tabz, accx, accy, accz,
             idx_v, g_v, fn_v, gsem, ssem):
    c = lax.axis_index("c")
    s = lax.axis_index("s")
    wid = c * NS + s
    base = s * TPS

    sl = pl.ds(base, TPS)
    stage = [
        pltpu.make_async_copy(verts_ref.at[0, 0, sl], tabx.at[sl], ssem.at[0]),
        pltpu.make_async_copy(verts_ref.at[1, 0, sl], taby.at[sl], ssem.at[0]),
        pltpu.make_async_copy(verts_ref.at[2, 0, sl], tabz.at[sl], ssem.at[0]),
        pltpu.make_async_copy(zeros_ref, accx.at[sl], ssem.at[0]),
        pltpu.make_async_copy(zeros_ref, accy.at[sl], ssem.at[0]),
        pltpu.make_async_copy(zeros_ref, accz.at[sl], ssem.at[0]),
    ] + [
        pltpu.make_async_copy(faces_ref.at[wid, d], idx_v.at[d], ssem.at[1])
        for d in range(3)
    ]
    for cp in stage:
        cp.start()
    for cp in stage:
        cp.wait()

    plsc.subcore_barrier()

    tabs = (tabx, taby, tabz)
    accs = (accx, accy, accz)

    def gather_cps(j, p):
        cps = []
        for k in range(3):
            idx = idx_v.at[k, j]
            for d in range(3):
                cps.append(pltpu.make_async_copy(
                    tabs[d].at[idx], g_v.at[p, k * 3 + d], gsem.at[p]))
        return cps

    def scatter_cps(j, p):
        cps = []
        for k in range(3):
            idx = idx_v.at[k, j]
            for d in range(3):
                cps.append(pltpu.make_async_copy(
                    fn_v.at[p, d], accs[d].at[idx], ssem.at[p]))
        return cps

    def compute(p):
        for i in range(CHUNK // 16):
            t = pl.ds(16 * i, 16)
            ax = g_v[p, 0, t]
            ay = g_v[p, 1, t]
            az = g_v[p, 2, t]
            bx = g_v[p, 3, t]
            by = g_v[p, 4, t]
            bz = g_v[p, 5, t]
            cx = g_v[p, 6, t]
            cy = g_v[p, 7, t]
            cz = g_v[p, 8, t]
            e0x = bx - ax
            e0y = by - ay
            e0z = bz - az
            e1x = cx - ax
            e1y = cy - ay
            e1z = cz - az
            fn_v[p, 0, t] = e0y * e1z - e0z * e1y
            fn_v[p, 1, t] = e0z * e1x - e0x * e1z
            fn_v[p, 2, t] = e0x * e1y - e0y * e1x

    for jj in range(NBUF - 1):
        for cp in gather_cps(jj, jj):
            cp.start()

    @pl.loop(0, CPW // NBUF)
    def _round(i):
        for half in range(NBUF):
            j = NBUF * i + half
            p = half

            @pl.when(j + NBUF - 1 < CPW)
            def _fire_next_gathers(j=j, p=p):
                for cp in gather_cps(j + NBUF - 1, (p + NBUF - 1) % NBUF):
                    cp.start()

            for cp in gather_cps(j, p):
                cp.wait()

            @pl.when(j >= NBUF)
            def _drain_prev_scatters(j=j, p=p):
                for cp in scatter_cps(j, p):
                    cp.wait()

            compute(p)
            for cp in scatter_cps(j, p):
                cp.start(add=True)

    for jj in range(NBUF):
        for cp in scatter_cps(CPW - NBUF + jj, jj):
            cp.wait()

    plsc.subcore_barrier()

    pltpu.sync_copy(accx.at[sl], part_ref.at[c, 0, 0, sl])
    pltpu.sync_copy(accy.at[sl], part_ref.at[c, 1, 0, sl])
    pltpu.sync_copy(accz.at[sl], part_ref.at[c, 2, 0, sl])


def _tc_finish_body(p_ref, o_ref):
    p = p_ref[...]
    vn = p[0] + p[1]
    sq = jnp.sum(vn * vn, axis=0, keepdims=True)
    norm = jnp.sqrt(sq)
    normalized = vn / jnp.maximum(norm, 1e-12)
    mask = sq > 1e-20
    default = jnp.where(
        lax.broadcasted_iota(jnp.int32, (3, 1), 0) == 2, 1.0, 0.0
    ).astype(jnp.float32)
    o_ref[...] = jnp.where(mask, normalized, default)


@jax.jit
def kernel(verts, faces):
    vertsT = jnp.zeros((3, 1, VP), jnp.float32).at[:, 0, :V].set(
        jnp.transpose(verts)
    )
    facesT = jnp.transpose(faces).astype(jnp.int32)
    facesP = (
        jnp.concatenate([facesT, jnp.zeros((3, FP - F), jnp.int32)], axis=1)
        .reshape(3, NW, CPW, CHUNK)
        .transpose(1, 0, 2, 3)
    )
    zeros = jnp.zeros((TPS,), jnp.float32)

    mesh = plsc.VectorSubcoreMesh(
        core_axis_name="c", subcore_axis_name="s",
        num_cores=NC, num_subcores=NS,
    )
    partial = pl.kernel(
        _sc_body,
        out_type=jax.ShapeDtypeStruct((NC, 3, 1, VP), jnp.float32),
        mesh=mesh,
        scratch_types=[
            pltpu.VMEM_SHARED((VP,), jnp.float32),
            pltpu.VMEM_SHARED((VP,), jnp.float32),
            pltpu.VMEM_SHARED((VP,), jnp.float32),
            pltpu.VMEM_SHARED((VP,), jnp.float32),
            pltpu.VMEM_SHARED((VP,), jnp.float32),
            pltpu.VMEM_SHARED((VP,), jnp.float32),
            pltpu.VMEM((3, CPW, CHUNK), jnp.int32),
            pltpu.VMEM((NBUF, 9, CHUNK), jnp.float32),
            pltpu.VMEM((NBUF, 3, CHUNK), jnp.float32),
            pltpu.SemaphoreType.DMA((NBUF,)),
            pltpu.SemaphoreType.DMA((NBUF,)),
        ],
    )(facesP, vertsT, zeros)
    partial = partial.reshape(NC, 3, VP)

    B = 2048
    out = pl.pallas_call(
        _tc_finish_body,
        grid=(VP // B,),
        in_specs=[pl.BlockSpec((NC, 3, B), lambda i: (0, 0, i))],
        out_specs=pl.BlockSpec((3, B), lambda i: (0, i)),
        out_shape=jax.ShapeDtypeStruct((3, VP), jnp.float32),
    )(partial)

    return jnp.transpose(out[:, :V])


# speedup vs baseline: 1.4876x; 1.0150x over previous
import jax
import jax.numpy as jnp
from jax import lax
from jax.experimental import pallas as pl
from jax.experimental.pallas import tpu as pltpu
from jax.experimental.pallas import tpu_sc as plsc

V = 100000
F = 200000
NC = 2
NS = 16
NW = NC * NS
CHUNK = 128
FP = 204800
CPW = FP // (NW * CHUNK)
VP = 100352
TPS = VP // NS
NBUF = 2


def _sc_body(faces_ref, verts_ref, zeros_ref, part_ref,
             tabx, taby, tabz, accx, accy, accz,
             idx_v, g_v, fn_v, gsem, ssem):
    c = lax.axis_index("c")
    s = lax.axis_index("s")
    wid = c * NS + s
    base = s * TPS

    sl = pl.ds(base, TPS)
    stage = [
        pltpu.make_async_copy(verts_ref.at[0, 0, sl], tabx.at[sl], ssem.at[0]),
        pltpu.make_async_copy(verts_ref.at[1, 0, sl], taby.at[sl], ssem.at[0]),
        pltpu.make_async_copy(verts_ref.at[2, 0, sl], tabz.at[sl], ssem.at[0]),
        pltpu.make_async_copy(zeros_ref, accx.at[sl], ssem.at[0]),
        pltpu.make_async_copy(zeros_ref, accy.at[sl], ssem.at[0]),
        pltpu.make_async_copy(zeros_ref, accz.at[sl], ssem.at[0]),
    ] + [
        pltpu.make_async_copy(faces_ref.at[d, wid], idx_v.at[d], ssem.at[1])
        for d in range(3)
    ]
    for cp in stage:
        cp.start()
    for cp in stage:
        cp.wait()

    plsc.subcore_barrier()

    tabs = (tabx, taby, tabz)
    accs = (accx, accy, accz)

    def gather_cps(j, p):
        cps = []
        for k in range(3):
            idx = idx_v.at[k, j, 0]
            for d in range(3):
                cps.append(pltpu.make_async_copy(
                    tabs[d].at[idx], g_v.at[p, k * 3 + d], gsem.at[p]))
        return cps

    def scatter_cps(j, p):
        cps = []
        for k in range(3):
            idx = idx_v.at[k, j, 0]
            for d in range(3):
                cps.append(pltpu.make_async_copy(
                    fn_v.at[p, d], accs[d].at[idx], ssem.at[p]))
        return cps

    def compute(p):
        for i in range(CHUNK // 16):
            t = pl.ds(16 * i, 16)
            ax = g_v[p, 0, t]
            ay = g_v[p, 1, t]
            az = g_v[p, 2, t]
            bx = g_v[p, 3, t]
            by = g_v[p, 4, t]
            bz = g_v[p, 5, t]
            cx = g_v[p, 6, t]
            cy = g_v[p, 7, t]
            cz = g_v[p, 8, t]
            e0x = bx - ax
            e0y = by - ay
            e0z = bz - az
            e1x = cx - ax
            e1y = cy - ay
            e1z = cz - az
            fn_v[p, 0, t] = e0y * e1z - e0z * e1y
            fn_v[p, 1, t] = e0z * e1x - e0x * e1z
            fn_v[p, 2, t] = e0x * e1y - e0y * e1x

    for jj in range(NBUF - 1):
        for cp in gather_cps(jj, jj):
            cp.start()

    @pl.loop(0, CPW // NBUF)
    def _round(i):
        for half in range(NBUF):
            j = NBUF * i + half
            p = half

            @pl.when(j + NBUF - 1 < CPW)
            def _fire_next_gathers(j=j, p=p):
                for cp in gather_cps(j + NBUF - 1, (p + NBUF - 1) % NBUF):
                    cp.start()

            for cp in gather_cps(j, p):
                cp.wait()

            @pl.when(j >= NBUF)
            def _drain_prev_scatters(j=j, p=p):
                for cp in scatter_cps(j, p):
                    cp.wait()

            compute(p)
            for cp in scatter_cps(j, p):
                cp.start(add=True)

    for jj in range(NBUF):
        for cp in scatter_cps(CPW - NBUF + jj, jj):
            cp.wait()

    plsc.subcore_barrier()

    pltpu.sync_copy(accx.at[sl], part_ref.at[c, 0, 0, sl])
    pltpu.sync_copy(accy.at[sl], part_ref.at[c, 1, 0, sl])
    pltpu.sync_copy(accz.at[sl], part_ref.at[c, 2, 0, sl])


def _tc_finish_body(p_ref, o_ref):
    p = p_ref[...]
    vn = p[0] + p[1]
    sq = jnp.sum(vn * vn, axis=0, keepdims=True)
    norm = jnp.sqrt(sq)
    normalized = vn / jnp.maximum(norm, 1e-12)
    mask = sq > 1e-20
    default = jnp.where(
        lax.broadcasted_iota(jnp.int32, (3, 1), 0) == 2, 1.0, 0.0
    ).astype(jnp.float32)
    o_ref[...] = jnp.where(mask, normalized, default)


@jax.jit
def kernel(verts, faces):
    vertsT = jnp.zeros((3, 1, VP), jnp.float32).at[:, 0, :V].set(
        jnp.transpose(verts)
    )
    facesT = jnp.transpose(faces).astype(jnp.int32)
    facesP = jnp.concatenate(
        [facesT, jnp.zeros((3, FP - F), jnp.int32)], axis=1
    ).reshape(3, NW, CPW, 1, CHUNK)
    zeros = jnp.zeros((TPS,), jnp.float32)

    mesh = plsc.VectorSubcoreMesh(
        core_axis_name="c", subcore_axis_name="s",
        num_cores=NC, num_subcores=NS,
    )
    partial = pl.kernel(
        _sc_body,
        out_type=jax.ShapeDtypeStruct((NC, 3, 1, VP), jnp.float32),
        mesh=mesh,
        scratch_types=[
            pltpu.VMEM_SHARED((VP,), jnp.float32),
            pltpu.VMEM_SHARED((VP,), jnp.float32),
            pltpu.VMEM_SHARED((VP,), jnp.float32),
            pltpu.VMEM_SHARED((VP,), jnp.float32),
            pltpu.VMEM_SHARED((VP,), jnp.float32),
            pltpu.VMEM_SHARED((VP,), jnp.float32),
            pltpu.VMEM((3, CPW, 1, CHUNK), jnp.int32),
            pltpu.VMEM((NBUF, 9, CHUNK), jnp.float32),
            pltpu.VMEM((NBUF, 3, CHUNK), jnp.float32),
            pltpu.SemaphoreType.DMA((NBUF,)),
            pltpu.SemaphoreType.DMA((NBUF,)),
        ],
    )(facesP, vertsT, zeros)
    partial = partial.reshape(NC, 3, VP)

    B = 2048
    out = pl.pallas_call(
        _tc_finish_body,
        grid=(VP // B,),
        in_specs=[pl.BlockSpec((NC, 3, B), lambda i: (0, 0, i))],
        out_specs=pl.BlockSpec((3, B), lambda i: (0, i)),
        out_shape=jax.ShapeDtypeStruct((3, VP), jnp.float32),
    )(partial)

    return jnp.transpose(out[:, :V])
